# Initial kernel scaffold; baseline (speedup 1.0000x reference)
import functools

import jax
import jax.numpy as jnp
from jax import lax
from jax.experimental import pallas as pl
from jax.experimental.pallas import tpu as pltpu
from jax.experimental.pallas import tpu_sc as plsc

_N_STIMULI = 30
_N_DIM = 10
_BATCH = 16384
_BETA = 3.0
_MIDPOINT = 0.5
_RATE = 5.0

_LANES = 16
_NUM_WORKERS = 32
_BPW = _BATCH // _NUM_WORKERS
_TROWS = 32
_TCOLS = 16


@functools.partial(
    pl.kernel,
    mesh=plsc.VectorSubcoreMesh(core_axis_name="c", subcore_axis_name="s"),
    out_type=jax.ShapeDtypeStruct((_BATCH,), jnp.float32),
    scratch_types=[
        pltpu.VMEM((_TROWS * _TCOLS,), jnp.float32),
        pltpu.VMEM((_N_DIM * _LANES,), jnp.float32),
        pltpu.VMEM((_BPW,), jnp.int32),
        pltpu.VMEM((_BPW,), jnp.int32),
        pltpu.VMEM((_BPW,), jnp.float32),
    ],
)
def _rate_sim_sc(tab_hbm, w_hbm, i_hbm, j_hbm, out_hbm,
                 tab_ref, w_ref, i_ref, j_ref, o_ref):
    nc = 2
    wid = lax.axis_index("s") * nc + lax.axis_index("c")
    base = wid * _BPW

    pltpu.sync_copy(tab_hbm, tab_ref)
    pltpu.sync_copy(w_hbm, w_ref)
    pltpu.sync_copy(i_hbm.at[pl.ds(base, _BPW)], i_ref)
    pltpu.sync_copy(j_hbm.at[pl.ds(base, _BPW)], j_ref)

    wvecs = [w_ref[pl.ds(d * _LANES, _LANES)] for d in range(_N_DIM)]

    for c in range(_BPW // _LANES):
        iv = i_ref[pl.ds(c * _LANES, _LANES)]
        jv = j_ref[pl.ds(c * _LANES, _LANES)]
        ia = iv * _TCOLS
        ja = jv * _TCOLS
        acc = jnp.zeros((_LANES,), jnp.float32)
        for d in range(_N_DIM):
            za = plsc.load_gather(tab_ref, [ia + d])
            zb = plsc.load_gather(tab_ref, [ja + d])
            df = za - zb
            acc = acc + wvecs[d] * df * df
        acc = jnp.maximum(acc, jnp.float32(1e-30))
        bits = lax.bitcast_convert_type(acc, jnp.int32)
        y = lax.bitcast_convert_type(
            jnp.int32(0x5F3759DF) - (bits >> 1), jnp.float32)
        for _ in range(3):
            y = y * (1.5 - 0.5 * acc * y * y)
        dist = acc * y
        s = jnp.exp(-_BETA * dist)
        prob = 1.0 / (1.0 + jnp.exp(_RATE * _MIDPOINT - _RATE * s))
        o_ref[pl.ds(c * _LANES, _LANES)] = prob

    pltpu.sync_copy(o_ref, out_hbm.at[pl.ds(base, _BPW)])


def kernel(inputs, table, w):
    i_arr = jnp.asarray(inputs[:, 0], jnp.int32)
    j_arr = jnp.asarray(inputs[:, 1], jnp.int32)
    tab = jnp.zeros((_TROWS, _TCOLS), jnp.float32)
    tab = tab.at[: _N_STIMULI + 1, : _N_DIM].set(table).reshape(-1)
    wb = jnp.broadcast_to(
        w.astype(jnp.float32)[:, None], (_N_DIM, _LANES)).reshape(-1)
    return _rate_sim_sc(tab, wb, i_arr, j_arr)


# speedup vs baseline: 5.9223x; 5.9223x over previous
import functools

import jax
import jax.numpy as jnp
from jax import lax
from jax.experimental import pallas as pl
from jax.experimental.pallas import tpu as pltpu
from jax.experimental.pallas import tpu_sc as plsc

_N_STIMULI = 30
_N_DIM = 10
_BATCH = 16384
_BETA = 3.0
_MIDPOINT = 0.5
_RATE = 5.0

_LANES = 16
_NUM_WORKERS = 32
_BPW = _BATCH // _NUM_WORKERS
_TROWS = 32
_TCOLS = 16


@functools.partial(
    pl.kernel,
    mesh=plsc.VectorSubcoreMesh(core_axis_name="c", subcore_axis_name="s"),
    compiler_params=pltpu.CompilerParams(needs_layout_passes=False),
    out_type=jax.ShapeDtypeStruct((_BATCH,), jnp.float32),
    scratch_types=[
        pltpu.VMEM((_TROWS * _TCOLS,), jnp.float32),
        pltpu.VMEM((_N_DIM * _LANES,), jnp.float32),
        pltpu.VMEM((_BPW,), jnp.int32),
        pltpu.VMEM((_BPW,), jnp.int32),
        pltpu.VMEM((_BPW,), jnp.float32),
    ],
)
def _rate_sim_sc(tab_hbm, w_hbm, i_hbm, j_hbm, out_hbm,
                 tab_ref, w_ref, i_ref, j_ref, o_ref):
    nc = 2
    wid = lax.axis_index("s") * nc + lax.axis_index("c")
    base = wid * _BPW

    pltpu.sync_copy(tab_hbm, tab_ref)
    pltpu.sync_copy(w_hbm, w_ref)
    pltpu.sync_copy(i_hbm.at[pl.ds(base, _BPW)], i_ref)
    pltpu.sync_copy(j_hbm.at[pl.ds(base, _BPW)], j_ref)

    wvecs = [w_ref[pl.ds(d * _LANES, _LANES)] for d in range(_N_DIM)]

    for c in range(_BPW // _LANES):
        iv = i_ref[pl.ds(c * _LANES, _LANES)]
        jv = j_ref[pl.ds(c * _LANES, _LANES)]
        ia = iv * _TCOLS
        ja = jv * _TCOLS
        acc = jnp.zeros((_LANES,), jnp.float32)
        for d in range(_N_DIM):
            za = plsc.load_gather(tab_ref, [ia + d])
            zb = plsc.load_gather(tab_ref, [ja + d])
            df = za - zb
            acc = acc + wvecs[d] * df * df
        acc = jnp.maximum(acc, jnp.float32(1e-30))
        bits = lax.bitcast_convert_type(acc, jnp.int32)
        y = lax.bitcast_convert_type(
            jnp.int32(0x5F3759DF) - (bits >> 1), jnp.float32)
        for _ in range(3):
            y = y * (1.5 - 0.5 * acc * y * y)
        dist = acc * y
        s = jnp.exp(-_BETA * dist)
        prob = 1.0 / (1.0 + jnp.exp(_RATE * _MIDPOINT - _RATE * s))
        o_ref[pl.ds(c * _LANES, _LANES)] = prob

    pltpu.sync_copy(o_ref, out_hbm.at[pl.ds(base, _BPW)])


def kernel(inputs, table, w):
    i_arr = jnp.asarray(inputs[:, 0], jnp.int32)
    j_arr = jnp.asarray(inputs[:, 1], jnp.int32)
    tab = jnp.zeros((_TROWS, _TCOLS), jnp.float32)
    tab = tab.at[: _N_STIMULI + 1, : _N_DIM].set(table).reshape(-1)
    wb = jnp.broadcast_to(
        w.astype(jnp.float32)[:, None], (_N_DIM, _LANES)).reshape(-1)
    return _rate_sim_sc(tab, wb, i_arr, j_arr)
